# baseline (speedup 1.0000x reference)
import functools

import jax
import jax.numpy as jnp
from jax import lax
from jax.experimental import pallas as pl
from jax.experimental.pallas import tpu as pltpu
from jax.experimental.pallas import tpu_sc as plsc

ROWS, COLS = 4096, 200
EMB_DIM = 64
B = ROWS * COLS
NC, NS = 2, 16
NW = NC * NS
B_PER_W = B // NW
CHUNK = 128
GROUP = 8
GROUP_ROWS = CHUNK * GROUP
N_GROUPS = B_PER_W // GROUP_ROWS
N_CHUNKS = B_PER_W // CHUNK


def _embed_body(x_hbm, table_hbm, out_hbm, idx_v, rows_v, gsem):
    wid = lax.axis_index("s") * NC + lax.axis_index("c")
    base = wid * B_PER_W
    pltpu.sync_copy(x_hbm.at[pl.ds(wid * N_CHUNKS, N_CHUNKS)], idx_v)

    def group(g, carry):
        copies = []
        for i in range(GROUP):
            copies.append(
                pltpu.async_copy(
                    table_hbm.at[idx_v.at[g * GROUP + i]],
                    rows_v.at[pl.ds(i * CHUNK, CHUNK)],
                    gsem,
                )
            )
        for c in copies:
            c.wait()
        pltpu.sync_copy(
            rows_v, out_hbm.at[pl.ds(base + g * GROUP_ROWS, GROUP_ROWS)]
        )
        return carry

    lax.fori_loop(0, N_GROUPS, group, 0)


@functools.partial(jax.jit, static_argnames=())
def _embed(x2d, table):
    mesh = plsc.VectorSubcoreMesh(core_axis_name="c", subcore_axis_name="s")
    k = pl.kernel(
        _embed_body,
        out_type=jax.ShapeDtypeStruct((B, EMB_DIM), jnp.float32),
        mesh=mesh,
        scratch_types=[
            pltpu.VMEM((N_CHUNKS, CHUNK), jnp.int32),
            pltpu.VMEM((GROUP_ROWS, EMB_DIM), jnp.float32),
            pltpu.SemaphoreType.DMA,
        ],
        compiler_params=pltpu.CompilerParams(use_tc_tiling_on_sc=False),
    )
    return k(x2d, table)


def kernel(x, table):
    x2d = x.reshape(B // CHUNK, CHUNK)
    out = _embed(x2d, table)
    return out.reshape(ROWS, COLS, EMB_DIM)


# speedup vs baseline: 1.0109x; 1.0109x over previous
import functools

import jax
import jax.numpy as jnp
from jax import lax
from jax.experimental import pallas as pl
from jax.experimental.pallas import tpu as pltpu
from jax.experimental.pallas import tpu_sc as plsc

ROWS, COLS = 4096, 200
EMB_DIM = 64
B = ROWS * COLS
NC, NS = 2, 16
NW = NC * NS
B_PER_W = B // NW
CHUNK = 128
N_CHUNKS = B_PER_W // CHUNK
NBUF = 8
N_OUTER = N_CHUNKS // NBUF


def _embed_body(x_hbm, table_hbm, out_hbm, idx_v, rows_v, *sems):
    wid = lax.axis_index("s") * NC + lax.axis_index("c")
    base = wid * B_PER_W
    pltpu.sync_copy(x_hbm.at[pl.ds(wid * N_CHUNKS, N_CHUNKS)], idx_v)

    def fire(c, slot):
        return pltpu.async_copy(
            table_hbm.at[idx_v.at[c]],
            rows_v.at[slot],
            sems[slot],
        )

    for b in range(NBUF - 1):
        fire(b, b)

    def outer(t, carry):
        for b in range(NBUF):
            c = t * NBUF + b
            nxt = c + NBUF - 1
            @pl.when(nxt < N_CHUNKS)
            def _():
                fire(nxt, (b + NBUF - 1) % NBUF)

            pltpu.make_async_copy(
                table_hbm.at[idx_v.at[c]], rows_v.at[b], sems[b]
            ).wait()
            pltpu.sync_copy(
                rows_v.at[b], out_hbm.at[pl.ds(base + c * CHUNK, CHUNK)]
            )
        return carry

    lax.fori_loop(0, N_OUTER, outer, 0)


@jax.jit
def _embed(x2d, table):
    mesh = plsc.VectorSubcoreMesh(core_axis_name="c", subcore_axis_name="s")
    k = pl.kernel(
        _embed_body,
        out_type=jax.ShapeDtypeStruct((B, EMB_DIM), jnp.float32),
        mesh=mesh,
        scratch_types=[
            pltpu.VMEM((N_CHUNKS, CHUNK), jnp.int32),
            pltpu.VMEM((NBUF, CHUNK, EMB_DIM), jnp.float32),
        ]
        + [pltpu.SemaphoreType.DMA] * NBUF,
        compiler_params=pltpu.CompilerParams(use_tc_tiling_on_sc=False),
    )
    return k(x2d, table)


def kernel(x, table):
    x2d = x.reshape(B // CHUNK, CHUNK)
    out = _embed(x2d, table)
    return out.reshape(ROWS, COLS, EMB_DIM)


# speedup vs baseline: 1.2366x; 1.2233x over previous
import jax
import jax.numpy as jnp
from jax import lax
from jax.experimental import pallas as pl
from jax.experimental.pallas import tpu as pltpu
from jax.experimental.pallas import tpu_sc as plsc

ROWS, COLS = 4096, 200
EMB_DIM = 64
PAD_DIM = 128
VOCAB = 1000000
B = ROWS * COLS
NC, NS = 2, 16
NW = NC * NS
B_PER_W = B // NW
CHUNK = 128
N_CHUNKS = B_PER_W // CHUNK
NBUF = 4
N_OUTER = N_CHUNKS // NBUF


def _gather_body(x_hbm, t2_hbm, out_hbm, idx_v, rows_v, *sems):
    wid = lax.axis_index("s") * NC + lax.axis_index("c")
    base = wid * B_PER_W
    pltpu.sync_copy(x_hbm.at[pl.ds(wid * N_CHUNKS, N_CHUNKS)], idx_v)

    def fire(c, slot):
        return pltpu.async_copy(
            t2_hbm.at[idx_v.at[c]], rows_v.at[slot], sems[slot]
        )

    for b in range(NBUF - 1):
        fire(b, b)

    def outer(t, carry):
        for b in range(NBUF):
            c = t * NBUF + b
            nxt = c + NBUF - 1

            @pl.when(nxt < N_CHUNKS)
            def _():
                fire(nxt, (b + NBUF - 1) % NBUF)

            pltpu.make_async_copy(
                t2_hbm.at[idx_v.at[c]], rows_v.at[b], sems[b]
            ).wait()
            pltpu.sync_copy(
                rows_v.at[b], out_hbm.at[pl.ds(base + c * CHUNK, CHUNK)]
            )
        return carry

    lax.fori_loop(0, N_OUTER, outer, 0)


@jax.jit
def _embed(x2d, t2):
    mesh = plsc.VectorSubcoreMesh(core_axis_name="c", subcore_axis_name="s")
    gather = pl.kernel(
        _gather_body,
        out_type=jax.ShapeDtypeStruct((B, PAD_DIM), jnp.float32),
        mesh=mesh,
        scratch_types=[
            pltpu.VMEM((N_CHUNKS, CHUNK), jnp.int32),
            pltpu.VMEM((NBUF, CHUNK, PAD_DIM), jnp.float32),
        ]
        + [pltpu.SemaphoreType.DMA] * NBUF,
        compiler_params=pltpu.CompilerParams(use_tc_tiling_on_sc=False),
    )
    return gather(x2d, t2)


def kernel(x, table):
    x2d = x.reshape(B // CHUNK, CHUNK)
    t2 = jnp.pad(table, ((0, 0), (0, PAD_DIM - EMB_DIM)))
    out = _embed(x2d, t2)
    return out[:, :EMB_DIM].reshape(ROWS, COLS, EMB_DIM)
